# Initial kernel scaffold; baseline (speedup 1.0000x reference)
import jax
import jax.numpy as jnp
from jax.experimental import pallas as pl


def kernel(x, edge_index, W1, b1, W2, b2, Wm1, bm1, gamma, beta, Wm2, bm2):
    raise NotImplementedError("write your pallas kernel here")


# speedup vs baseline: 4.0096x; 4.0096x over previous
import functools

import jax
import jax.numpy as jnp
from jax import lax
from jax.experimental import pallas as pl
from jax.experimental.pallas import tpu as pltpu
from jax.experimental.pallas import tpu_sc as plsc

NC = 2
NS = 16
NW = NC * NS
B = 128

F32 = jnp.float32


def _mesh():
  return plsc.VectorSubcoreMesh(core_axis_name="c", subcore_axis_name="s")


def _fill(buf, nrows, ncols, value):
  v = jnp.full((16,), value, F32)
  per_row = ncols // 16

  def body(i, _):
    buf[i // per_row, pl.ds((i % per_row) * 16, 16)] = v
    return 0

  lax.fori_loop(0, nrows * per_row, body, 0)


def _deg_body(nb, rows_per_tile,
              src2, dst2, zeros_hbm, deg_s_out, deg_d_out,
              idx_s, idx_d, ones_v, acc_s, acc_d):
  c = lax.axis_index("c")
  s = lax.axis_index("s")
  w = c * NS + s

  _fill(ones_v, B, 16, 1.0)

  base = s * rows_per_tile
  pltpu.sync_copy(zeros_hbm, acc_s.at[pl.ds(base, rows_per_tile)])
  pltpu.sync_copy(zeros_hbm, acc_d.at[pl.ds(base, rows_per_tile)])
  plsc.subcore_barrier()

  pltpu.sync_copy(src2.at[pl.ds(w * nb, nb)], idx_s)
  pltpu.sync_copy(dst2.at[pl.ds(w * nb, nb)], idx_d)

  def edge_body(j, _):
    pltpu.sync_copy(ones_v, acc_s.at[idx_s.at[j]], add=True)
    pltpu.sync_copy(ones_v, acc_d.at[idx_d.at[j]], add=True)
    return 0

  lax.fori_loop(0, nb, edge_body, 0)
  plsc.subcore_barrier()

  pltpu.sync_copy(acc_s.at[pl.ds(base, rows_per_tile)],
                  deg_s_out.at[c, pl.ds(base, rows_per_tile)])
  pltpu.sync_copy(acc_d.at[pl.ds(base, rows_per_tile)],
                  deg_d_out.at[c, pl.ds(base, rows_per_tile)])


def _deg_call(np_pad, nb, src2, dst2, zeros_hbm):
  rows_per_tile = np_pad // NS
  out = jax.ShapeDtypeStruct((NC, np_pad, 16), F32)
  f = pl.kernel(
      functools.partial(_deg_body, nb, rows_per_tile),
      out_type=[out, out],
      mesh=_mesh(),
      scratch_types=[
          pltpu.VMEM((nb, B), jnp.int32),
          pltpu.VMEM((nb, B), jnp.int32),
          pltpu.VMEM((B, 16), F32),
          pltpu.VMEM_SHARED((np_pad, 16), F32),
          pltpu.VMEM_SHARED((np_pad, 16), F32),
      ],
  )
  return f(src2, dst2, zeros_hbm)


def _prop_body(nb, rows_per_tile,
               table, src2, dst2, zeros_hbm, part_out,
               idx_s, idx_d, rows, acc):
  c = lax.axis_index("c")
  s = lax.axis_index("s")
  w = c * NS + s

  base = s * rows_per_tile
  pltpu.sync_copy(zeros_hbm, acc.at[pl.ds(base, rows_per_tile)])
  plsc.subcore_barrier()

  pltpu.sync_copy(src2.at[pl.ds(w * nb, nb)], idx_s)
  pltpu.sync_copy(dst2.at[pl.ds(w * nb, nb)], idx_d)

  def edge_body(j, _):
    pltpu.sync_copy(table.at[idx_s.at[j]], rows.at[0])
    pltpu.sync_copy(rows.at[0], acc.at[idx_d.at[j]], add=True)
    return 0

  lax.fori_loop(0, nb, edge_body, 0)
  plsc.subcore_barrier()

  pltpu.sync_copy(acc.at[pl.ds(base, rows_per_tile)],
                  part_out.at[c, pl.ds(base, rows_per_tile)])


def _prop_call(np_pad, nb, table, src2, dst2, zeros_hbm):
  rows_per_tile = np_pad // NS
  f = pl.kernel(
      functools.partial(_prop_body, nb, rows_per_tile),
      out_type=jax.ShapeDtypeStruct((NC, np_pad, 128), F32),
      mesh=_mesh(),
      scratch_types=[
          pltpu.VMEM((nb, B), jnp.int32),
          pltpu.VMEM((nb, B), jnp.int32),
          pltpu.VMEM((1, B, 128), F32),
          pltpu.VMEM_SHARED((np_pad, 128), F32),
      ],
  )
  return f(table, src2, dst2, zeros_hbm)


def _tc1_body(xp_ref, ds_ref, dd_ref, w1_ref, hw1_ref, ns_ref, nd_ref):
  deg_s = ds_ref[0, :, 0:1] + ds_ref[1, :, 0:1] + 1.0
  deg_d = dd_ref[0, :, 0:1] + dd_ref[1, :, 0:1] + 1.0
  ns = lax.rsqrt(deg_s)
  nd = lax.rsqrt(deg_d)
  ns_ref[...] = ns
  nd_ref[...] = nd
  xw = jnp.dot(xp_ref[...], w1_ref[...], preferred_element_type=F32)
  hw1_ref[...] = xw * ns


def _tc2_body(part_ref, hw1_ref, ns_ref, nd_ref, b1_ref, w2_ref, hw2_ref):
  m = part_ref[0] + part_ref[1] + hw1_ref[...]
  h = jax.nn.relu(m * nd_ref[...] + b1_ref[...][None, :])
  hw2_ref[...] = jnp.dot(h, w2_ref[...],
                         preferred_element_type=F32) * ns_ref[...]


def _tc3_body(part_ref, hw2_ref, nd_ref, b2_ref, wm1_ref, bm1_ref,
              gamma_ref, beta_ref, wm2_ref, bm2_ref, out_ref):
  m = part_ref[0] + part_ref[1] + hw2_ref[...]
  h = jax.nn.relu(m * nd_ref[...] + b2_ref[...][None, :])
  t = jax.nn.relu(
      jnp.dot(h, wm1_ref[...], preferred_element_type=F32)
      + bm1_ref[...][None, :])
  inv = 1.0 / jnp.sqrt(jnp.float32(1.0 + 1e-5))
  t = t * (gamma_ref[...] * inv)[None, :] + beta_ref[...][None, :]
  out_ref[...] = (jnp.dot(t, wm2_ref[...], preferred_element_type=F32)
                  + bm2_ref[...][None, :])


def _tc_call(body, out_shape, *args):
  return pl.pallas_call(body, out_shape=out_shape)(*args)


def kernel(x, edge_index, W1, b1, W2, b2, Wm1, bm1, gamma, beta, Wm2, bm2):
  n, _ = x.shape
  e = edge_index.shape[1]
  np_pad = -(-(n + 1) // 128) * 128
  nb = -(-e // (NW * B))
  nb = -(-nb // 8) * 8
  e_pad = NW * nb * B

  ei = jnp.pad(edge_index, ((0, 0), (0, e_pad - e)), constant_values=n)
  src2 = ei[0].reshape(-1, B)
  dst2 = ei[1].reshape(-1, B)
  xp = jnp.pad(x, ((0, np_pad - n), (0, 0)))
  zeros_hbm = jnp.zeros((np_pad // NS, 128), F32)
  zeros16_hbm = jnp.zeros((np_pad // NS, 16), F32)

  deg_s, deg_d = _deg_call(np_pad, nb, src2, dst2, zeros16_hbm)

  hw1, ns, nd = _tc_call(
      _tc1_body,
      [jax.ShapeDtypeStruct((np_pad, 128), F32),
       jax.ShapeDtypeStruct((np_pad, 1), F32),
       jax.ShapeDtypeStruct((np_pad, 1), F32)],
      xp, deg_s, deg_d, W1)

  part1 = _prop_call(np_pad, nb, hw1, src2, dst2, zeros_hbm)

  hw2 = _tc_call(
      _tc2_body,
      jax.ShapeDtypeStruct((np_pad, 128), F32),
      part1, hw1, ns, nd, b1, W2)

  part2 = _prop_call(np_pad, nb, hw2, src2, dst2, zeros_hbm)

  out_full = _tc_call(
      _tc3_body,
      jax.ShapeDtypeStruct((np_pad, 2), F32),
      part2, hw2, nd, b2, Wm1, bm1, gamma, beta, Wm2, bm2)

  return out_full[:n]
